# baseline (speedup 1.0000x reference)
import functools
import math

import jax
import jax.numpy as jnp
from jax import lax
from jax.experimental import pallas as pl
from jax.experimental.pallas import tpu as pltpu
from jax.experimental.pallas import tpu_sc as plsc

VOCAB = 50257
D_MODEL = 768
BATCH = 4
SEQ = 2048

NC = 2
NS = 16
LANES = 16
NW = NC * NS
NTOK = BATCH * SEQ
POS_PER_W = SEQ // NW
SP = 16
NSC = POS_PER_W // SP
QROWS = BATCH * SP
VECS_PER_ROW = D_MODEL // LANES
SCALE = math.sqrt(D_MODEL)

_mesh = plsc.VectorSubcoreMesh(core_axis_name="c", subcore_axis_name="s")


@functools.partial(
    pl.kernel,
    out_type=jax.ShapeDtypeStruct((NTOK, D_MODEL), jnp.float32),
    mesh=_mesh,
    scratch_types=[
        pltpu.VMEM((NSC, QROWS), jnp.int32),
        pltpu.VMEM((QROWS, D_MODEL), jnp.float32),
        pltpu.VMEM((QROWS, D_MODEL), jnp.float32),
        pltpu.VMEM((SP, D_MODEL), jnp.float32),
        pltpu.VMEM((SP, D_MODEL), jnp.float32),
        pltpu.SemaphoreType.DMA,
        pltpu.SemaphoreType.DMA,
        pltpu.SemaphoreType.DMA,
        pltpu.SemaphoreType.DMA,
        pltpu.SemaphoreType.DMA,
        pltpu.SemaphoreType.DMA,
    ],
)
def _emb_kernel(ids_hbm, tok_hbm, pos_hbm, out_hbm,
                idx_v, q0, q1, p0, p1, gs0, gs1, ps0, ps1, ws0, ws1):
    wid = lax.axis_index("s") * NC + lax.axis_index("c")
    s_base = wid * POS_PER_W
    quads = [q0, q1]
    poss = [p0, p1]
    gsems = [gs0, gs1]
    psems = [ps0, ps1]
    wsems = [ws0, ws1]

    pltpu.sync_copy(ids_hbm.at[wid], idx_v)

    def issue(sc):
        bu = sc % 2
        g = pltpu.async_copy(tok_hbm.at[idx_v.at[sc]], quads[bu], gsems[bu])
        p = pltpu.async_copy(
            pos_hbm.at[pl.ds(s_base + sc * SP, SP)], poss[bu], psems[bu])
        return g, p

    gathers = [None] * NSC
    pos_cps = [None] * NSC
    writes = [[None] * BATCH for _ in range(NSC)]
    gathers[0], pos_cps[0] = issue(0)

    for sc in range(NSC):
        bu = sc % 2
        if sc + 1 < NSC:
            if sc >= 1:
                for wcp in writes[sc - 1]:
                    wcp.wait()
            gathers[sc + 1], pos_cps[sc + 1] = issue(sc + 1)
        gathers[sc].wait()
        pos_cps[sc].wait()

        def row_body(r, _, bu=bu):
            q = quads[bu]
            pv_ref = poss[bu]
            for l in range(VECS_PER_ROW):
                sl = pl.ds(l * LANES, LANES)
                pv = pv_ref[r, sl]
                for b in range(BATCH):
                    q[b * SP + r, sl] = q[b * SP + r, sl] * SCALE + pv
            return _

        lax.fori_loop(0, SP, row_body, 0, unroll=False)

        for b in range(BATCH):
            writes[sc][b] = pltpu.async_copy(
                quads[bu].at[pl.ds(b * SP, SP)],
                out_hbm.at[pl.ds(b * SEQ + s_base + sc * SP, SP)],
                wsems[bu])

    for sc in (NSC - 2, NSC - 1):
        for wcp in writes[sc]:
            wcp.wait()


def kernel(token_ids, tok_table, pos_table):
    ids = jnp.reshape(token_ids.astype(jnp.int32), (BATCH, NW, NSC, SP))
    ids = jnp.transpose(ids, (1, 2, 0, 3)).reshape(NW, NSC, QROWS)
    out = _emb_kernel(ids, tok_table, pos_table)
    return jnp.reshape(out, (BATCH, SEQ, D_MODEL))


# speedup vs baseline: 1.2102x; 1.2102x over previous
import functools
import math

import jax
import jax.numpy as jnp
from jax import lax
from jax.experimental import pallas as pl
from jax.experimental.pallas import tpu as pltpu
from jax.experimental.pallas import tpu_sc as plsc

VOCAB = 50257
D_MODEL = 768
BATCH = 4
SEQ = 2048

NC = 2
NS = 16
LANES = 16
NW = NC * NS
NTOK = BATCH * SEQ
POS_PER_W = SEQ // NW
SP = 16
NSC = POS_PER_W // SP
QROWS = BATCH * SP
VECS_PER_ROW = D_MODEL // LANES
SCALE = math.sqrt(D_MODEL)

_mesh = plsc.VectorSubcoreMesh(core_axis_name="c", subcore_axis_name="s")


@functools.partial(
    pl.kernel,
    out_type=jax.ShapeDtypeStruct((NTOK, D_MODEL), jnp.float32),
    mesh=_mesh,
    scratch_types=[
        pltpu.VMEM((NSC, QROWS), jnp.int32),
        pltpu.VMEM((QROWS, D_MODEL), jnp.float32),
        pltpu.VMEM((QROWS, D_MODEL), jnp.float32),
        pltpu.VMEM((SP, D_MODEL), jnp.float32),
        pltpu.VMEM((SP, D_MODEL), jnp.float32),
        pltpu.SemaphoreType.DMA,
        pltpu.SemaphoreType.DMA,
        pltpu.SemaphoreType.DMA,
        pltpu.SemaphoreType.DMA,
        pltpu.SemaphoreType.DMA,
        pltpu.SemaphoreType.DMA,
    ],
)
def _emb_kernel(ids_hbm, tok_hbm, pos_hbm, out_hbm,
                idx_v, q0, q1, p0, p1, gs0, gs1, ps0, ps1, ws0, ws1):
    wid = lax.axis_index("s") * NC + lax.axis_index("c")
    s_base = wid * POS_PER_W
    quads = [q0, q1]
    poss = [p0, p1]
    gsems = [gs0, gs1]
    psems = [ps0, ps1]
    wsems = [ws0, ws1]

    pltpu.sync_copy(ids_hbm.at[wid], idx_v)

    def issue(sc):
        bu = sc % 2
        g = pltpu.async_copy(tok_hbm.at[idx_v.at[sc]], quads[bu], gsems[bu])
        p = pltpu.async_copy(
            pos_hbm.at[pl.ds(s_base + sc * SP, SP)], poss[bu], psems[bu])
        return g, p

    gathers = [None] * NSC
    pos_cps = [None] * NSC
    writes = [[None] * BATCH for _ in range(NSC)]
    gathers[0], pos_cps[0] = issue(0)

    for sc in range(NSC):
        bu = sc % 2
        if sc + 1 < NSC:
            if sc >= 1:
                for wcp in writes[sc - 1]:
                    wcp.wait()
            gathers[sc + 1], pos_cps[sc + 1] = issue(sc + 1)
        gathers[sc].wait()
        pos_cps[sc].wait()

        def row_body(r, bu=bu):
            q = quads[bu]
            pv_ref = poss[bu]
            for l in range(VECS_PER_ROW):
                sl = pl.ds(l * LANES, LANES)
                pv = pv_ref[r, sl]
                for b in range(BATCH):
                    q[b * SP + r, sl] = q[b * SP + r, sl] * SCALE + pv

        plsc.parallel_loop(0, SP)(row_body)

        for b in range(BATCH):
            writes[sc][b] = pltpu.async_copy(
                quads[bu].at[pl.ds(b * SP, SP)],
                out_hbm.at[pl.ds(b * SEQ + s_base + sc * SP, SP)],
                wsems[bu])

    for sc in (NSC - 2, NSC - 1):
        for wcp in writes[sc]:
            wcp.wait()


def kernel(token_ids, tok_table, pos_table):
    ids = jnp.reshape(token_ids.astype(jnp.int32), (BATCH, NW, NSC, SP))
    ids = jnp.transpose(ids, (1, 2, 0, 3)).reshape(NW, NSC, QROWS)
    out = _emb_kernel(ids, tok_table, pos_table)
    return jnp.reshape(out, (BATCH, SEQ, D_MODEL))


# speedup vs baseline: 1.6822x; 1.3900x over previous
import functools
import math

import jax
import jax.numpy as jnp
from jax import lax
from jax.experimental import pallas as pl
from jax.experimental.pallas import tpu as pltpu
from jax.experimental.pallas import tpu_sc as plsc

VOCAB = 50257
D_MODEL = 768
BATCH = 4
SEQ = 2048

NC = 2
NS = 16
LANES = 16
NW = NC * NS
NTOK = BATCH * SEQ
POS_PER_W = SEQ // NW
SP = 8
NSC = POS_PER_W // SP
QROWS = BATCH * SP
NBUF = 3
VECS_PER_ROW = D_MODEL // LANES
SCALE = math.sqrt(D_MODEL)

_mesh = plsc.VectorSubcoreMesh(core_axis_name="c", subcore_axis_name="s")


@functools.partial(
    pl.kernel,
    out_type=jax.ShapeDtypeStruct((NTOK, D_MODEL), jnp.float32),
    mesh=_mesh,
    scratch_types=[
        pltpu.VMEM((NSC, QROWS), jnp.int32),
        pltpu.VMEM((POS_PER_W, D_MODEL), jnp.float32),
        pltpu.VMEM((QROWS, D_MODEL), jnp.float32),
        pltpu.VMEM((QROWS, D_MODEL), jnp.float32),
        pltpu.VMEM((QROWS, D_MODEL), jnp.float32),
        pltpu.SemaphoreType.DMA,
        pltpu.SemaphoreType.DMA,
        pltpu.SemaphoreType.DMA,
        pltpu.SemaphoreType.DMA,
        pltpu.SemaphoreType.DMA,
        pltpu.SemaphoreType.DMA,
        pltpu.SemaphoreType.DMA,
    ],
)
def _emb_kernel(ids_hbm, tok_hbm, pos_hbm, out_hbm,
                idx_v, pos_v, q0, q1, q2, psem, gs0, gs1, gs2, ws0, ws1, ws2):
    wid = lax.axis_index("s") * NC + lax.axis_index("c")
    s_base = wid * POS_PER_W
    quads = [q0, q1, q2]
    gsems = [gs0, gs1, gs2]
    wsems = [ws0, ws1, ws2]

    pltpu.sync_copy(ids_hbm.at[wid], idx_v)
    pos_cp = pltpu.async_copy(
        pos_hbm.at[pl.ds(s_base, POS_PER_W)], pos_v, psem)

    def issue_gather(sc):
        bu = sc % NBUF
        return pltpu.async_copy(tok_hbm.at[idx_v.at[sc]], quads[bu], gsems[bu])

    gathers = [None] * NSC
    writes = [[None] * BATCH for _ in range(NSC)]
    for sc in range(NBUF - 1):
        gathers[sc] = issue_gather(sc)
    pos_cp.wait()

    for sc in range(NSC):
        bu = sc % NBUF
        nxt = sc + NBUF - 1
        if nxt < NSC:
            if nxt >= NBUF:
                for wcp in writes[nxt - NBUF]:
                    wcp.wait()
            gathers[nxt] = issue_gather(nxt)
        gathers[sc].wait()

        def vec_body(i, sc=sc, bu=bu):
            r = i & (SP - 1)
            l = i >> 3
            sl = pl.ds(l * LANES, LANES)
            pv = pos_v[sc * SP + r, sl]
            q = quads[bu]
            for b in range(BATCH):
                q[b * SP + r, sl] = q[b * SP + r, sl] * SCALE + pv

        plsc.parallel_loop(0, SP * VECS_PER_ROW, unroll=2)(vec_body)

        for b in range(BATCH):
            writes[sc][b] = pltpu.async_copy(
                quads[bu].at[pl.ds(b * SP, SP)],
                out_hbm.at[pl.ds(b * SEQ + s_base + sc * SP, SP)],
                wsems[bu])

    for sc in range(NSC - NBUF, NSC):
        for wcp in writes[sc]:
            wcp.wait()


def kernel(token_ids, tok_table, pos_table):
    ids = jnp.reshape(token_ids.astype(jnp.int32), (BATCH, NW, NSC, SP))
    ids = jnp.transpose(ids, (1, 2, 0, 3)).reshape(NW, NSC, QROWS)
    out = _emb_kernel(ids, tok_table, pos_table)
    return jnp.reshape(out, (BATCH, SEQ, D_MODEL))


# speedup vs baseline: 1.6898x; 1.0045x over previous
import functools
import math

import jax
import jax.numpy as jnp
from jax import lax
from jax.experimental import pallas as pl
from jax.experimental.pallas import tpu as pltpu
from jax.experimental.pallas import tpu_sc as plsc

VOCAB = 50257
D_MODEL = 768
BATCH = 4
SEQ = 2048

NC = 2
NS = 16
LANES = 16
NW = NC * NS
NTOK = BATCH * SEQ
POS_PER_W = SEQ // NW
SP = 8
NSC = POS_PER_W // SP
QROWS = BATCH * SP
NBUF = 4
LOOK = 2
VECS_PER_ROW = D_MODEL // LANES
SCALE = math.sqrt(D_MODEL)

_mesh = plsc.VectorSubcoreMesh(core_axis_name="c", subcore_axis_name="s")


@functools.partial(
    pl.kernel,
    out_type=jax.ShapeDtypeStruct((NTOK, D_MODEL), jnp.float32),
    mesh=_mesh,
    scratch_types=[
        pltpu.VMEM((NSC, QROWS), jnp.int32),
        pltpu.VMEM((SP, D_MODEL), jnp.float32),
        pltpu.VMEM((SP, D_MODEL), jnp.float32),
        pltpu.VMEM((QROWS, D_MODEL), jnp.float32),
        pltpu.VMEM((QROWS, D_MODEL), jnp.float32),
        pltpu.VMEM((QROWS, D_MODEL), jnp.float32),
        pltpu.VMEM((QROWS, D_MODEL), jnp.float32),
        pltpu.SemaphoreType.DMA,
        pltpu.SemaphoreType.DMA,
        pltpu.SemaphoreType.DMA,
        pltpu.SemaphoreType.DMA,
        pltpu.SemaphoreType.DMA,
        pltpu.SemaphoreType.DMA,
        pltpu.SemaphoreType.DMA,
        pltpu.SemaphoreType.DMA,
        pltpu.SemaphoreType.DMA,
        pltpu.SemaphoreType.DMA,
    ],
)
def _emb_kernel(ids_hbm, tok_hbm, pos_hbm, out_hbm,
                idx_v, pv0, pv1, q0, q1, q2, q3,
                ps0, ps1, gs0, gs1, gs2, gs3, ws0, ws1, ws2, ws3):
    wid = lax.axis_index("s") * NC + lax.axis_index("c")
    s_base = wid * POS_PER_W
    poss = [pv0, pv1]
    psems = [ps0, ps1]
    quads = [q0, q1, q2, q3]
    gsems = [gs0, gs1, gs2, gs3]
    wsems = [ws0, ws1, ws2, ws3]

    pltpu.sync_copy(ids_hbm.at[wid], idx_v)

    def issue_gather(sc):
        bu = sc % NBUF
        return pltpu.async_copy(tok_hbm.at[idx_v.at[sc]], quads[bu], gsems[bu])

    def issue_pos(sc):
        pb = sc % 2
        return pltpu.async_copy(
            pos_hbm.at[pl.ds(s_base + sc * SP, SP)], poss[pb], psems[pb])

    gathers = [None] * NSC
    pos_cps = [None] * NSC
    writes = [[None] * BATCH for _ in range(NSC)]
    for sc in range(LOOK):
        gathers[sc] = issue_gather(sc)
        pos_cps[sc] = issue_pos(sc)

    for sc in range(NSC):
        bu = sc % NBUF
        pb = sc % 2
        nxt = sc + LOOK
        if nxt < NSC:
            if nxt >= NBUF:
                for wcp in writes[nxt - NBUF]:
                    wcp.wait()
            gathers[nxt] = issue_gather(nxt)
            pos_cps[nxt] = issue_pos(nxt)
        gathers[sc].wait()
        pos_cps[sc].wait()

        def vec_body(i, bu=bu, pb=pb):
            r = i & (SP - 1)
            l = i >> 3
            sl = pl.ds(l * LANES, LANES)
            pv = poss[pb][r, sl]
            q = quads[bu]
            for b in range(BATCH):
                q[b * SP + r, sl] = q[b * SP + r, sl] * SCALE + pv

        plsc.parallel_loop(0, SP * VECS_PER_ROW, unroll=2)(vec_body)

        for b in range(BATCH):
            writes[sc][b] = pltpu.async_copy(
                quads[bu].at[pl.ds(b * SP, SP)],
                out_hbm.at[pl.ds(b * SEQ + s_base + sc * SP, SP)],
                wsems[bu])

    for sc in range(NSC - LOOK - 2, NSC):
        if sc >= 0:
            for wcp in writes[sc]:
                wcp.wait()


def kernel(token_ids, tok_table, pos_table):
    ids = jnp.reshape(token_ids.astype(jnp.int32), (BATCH, NW, NSC, SP))
    ids = jnp.transpose(ids, (1, 2, 0, 3)).reshape(NW, NSC, QROWS)
    out = _emb_kernel(ids, tok_table, pos_table)
    return jnp.reshape(out, (BATCH, SEQ, D_MODEL))
